# baseline (speedup 1.0000x reference)
import functools

import jax
import jax.numpy as jnp
from jax import lax
from jax.experimental import pallas as pl
from jax.experimental.pallas import tpu as pltpu
from jax.experimental.pallas import tpu_sc as plsc

NC = 2
NS = 16
NW = NC * NS
LANES = 16
CHUNK = 40
NBUF = 5
N_PAD = 10240
DEGW = 32
D = 128


def _fill_rows(ref, n_rows, row_words, value):
  assert row_words % LANES == 0
  per_row = row_words // LANES

  def body(r, carry):
    for c in range(per_row):
      ref[r, pl.ds(c * LANES, LANES)] = jnp.full((LANES,), value, jnp.float32)
    return carry

  lax.fori_loop(0, n_rows, body, 0)


def _zero_shared(zbuf, acc_sh, s, pt, width):
  zr = zbuf.shape[0]
  _fill_rows(zbuf, zr, width, 0.0)
  for j in range(pt // zr):
    pltpu.sync_copy(zbuf, acc_sh.at[pl.ds(s * pt + j * zr, zr)])


def _sc_deg_body(n_pad, k, dst1_hbm, out_hbm, didx, ones_v, zbuf,
                 deg_sh, *sems):
  c = lax.axis_index("c")
  s = lax.axis_index("s")
  w = s * NC + c
  pt = n_pad // NS
  pe = k * CHUNK
  ssems = sems[:NBUF]
  isems = sems[NBUF:]

  _fill_rows(ones_v, CHUNK, DEGW, 1.0)
  _zero_shared(zbuf, deg_sh, s, pt, DEGW)
  plsc.subcore_barrier()

  assert k % NBUF == 0

  def load_idx(j, b):
    pltpu.async_copy(
        dst1_hbm.at[pl.ds(w * pe + j * CHUNK, CHUNK)], didx.at[b], isems[b])

  def scatter(j, b):
    pltpu.make_async_copy(
        dst1_hbm.at[pl.ds(w * pe + j * CHUNK, CHUNK)],
        didx.at[b], isems[b]).wait()
    pltpu.async_copy(ones_v, deg_sh.at[didx.at[b]], ssems[b], add=True)

  for b in range(NBUF):
    load_idx(b, b)
  for b in range(NBUF):
    scatter(b, b)

  def body(grp, carry):
    base = grp * NBUF
    for b in range(NBUF):
      pltpu.make_async_copy(
          ones_v, deg_sh.at[didx.at[b]], ssems[b]).wait()
      load_idx(base + b, b)
    for b in range(NBUF):
      scatter(base + b, b)
    return carry

  lax.fori_loop(1, k // NBUF, body, 0)
  for b in range(NBUF):
    pltpu.make_async_copy(
        ones_v, deg_sh.at[didx.at[b]], ssems[b]).wait()
  plsc.subcore_barrier()
  pltpu.sync_copy(deg_sh.at[pl.ds(s * pt, pt)],
                  out_hbm.at[c, pl.ds(s * pt, pt)])


def _sc_agg_body(n_pad, k, g_hbm, src1_hbm, dst1_hbm, out_hbm,
                 sidx_1d, didx, rows, zbuf, acc_sh, *sems):
  c = lax.axis_index("c")
  s = lax.axis_index("s")
  w = s * NC + c
  pt = n_pad // NS
  pe = k * CHUNK
  gsems = sems[:NBUF]
  ssems = sems[NBUF:2 * NBUF]
  isems = sems[2 * NBUF:]

  _zero_shared(zbuf, acc_sh, s, pt, D)
  pltpu.sync_copy(src1_hbm.at[pl.ds(w * pe, pe)], sidx_1d)
  plsc.subcore_barrier()

  assert k % NBUF == 0

  def fire(j, b):
    pltpu.async_copy(
        dst1_hbm.at[pl.ds(w * pe + j * CHUNK, CHUNK)], didx.at[b], isems[b])
    return pltpu.async_copy(
        g_hbm.at[sidx_1d.at[pl.ds(j * CHUNK, CHUNK)]], rows.at[b], gsems[b])

  def scatter(j, b, gd):
    gd.wait()
    pltpu.make_async_copy(
        dst1_hbm.at[pl.ds(w * pe + j * CHUNK, CHUNK)],
        didx.at[b], isems[b]).wait()
    pltpu.async_copy(
        rows.at[b], acc_sh.at[didx.at[b]], ssems[b], add=True)

  gds = [fire(b, b) for b in range(NBUF)]
  for b in range(NBUF):
    scatter(b, b, gds[b])

  def body(grp, carry):
    base = grp * NBUF
    gds = []
    for b in range(NBUF):
      pltpu.make_async_copy(
          rows.at[b], acc_sh.at[didx.at[b]], ssems[b]).wait()
      gds.append(fire(base + b, b))
    for b in range(NBUF):
      scatter(base + b, b, gds[b])
    return carry

  lax.fori_loop(1, k // NBUF, body, 0)
  for b in range(NBUF):
    pltpu.make_async_copy(
        rows.at[b], acc_sh.at[didx.at[b]], ssems[b]).wait()
  plsc.subcore_barrier()
  pltpu.sync_copy(acc_sh.at[pl.ds(s * pt, pt)],
                  out_hbm.at[c, pl.ds(s * pt, pt)])


def _sc_mesh():
  return plsc.VectorSubcoreMesh(core_axis_name="c", subcore_axis_name="s",
                                num_cores=NC, num_subcores=NS)


def _sc_deg(dst1):
  e = dst1.shape[0]
  assert e % (CHUNK * NW) == 0
  k = e // (CHUNK * NW)
  pt = N_PAD // NS
  kern = pl.kernel(
      functools.partial(_sc_deg_body, N_PAD, k),
      out_type=jax.ShapeDtypeStruct((NC, N_PAD, DEGW), jnp.float32),
      mesh=_sc_mesh(),
      scratch_types=[
          pltpu.VMEM((NBUF, CHUNK), jnp.int32),
          pltpu.VMEM((CHUNK, DEGW), jnp.float32),
          pltpu.VMEM((pt // 10, DEGW), jnp.float32),
          pltpu.VMEM_SHARED((N_PAD, DEGW), jnp.float32),
      ] + [pltpu.SemaphoreType.DMA] * (2 * NBUF),
      compiler_params=pltpu.CompilerParams(use_tc_tiling_on_sc=False),
  )
  return kern(dst1)


def _sc_agg(g, src1, dst1):
  n = g.shape[0]
  e = src1.shape[0]
  assert n <= N_PAD and g.shape[1] == D and e % (CHUNK * NW) == 0
  k = e // (CHUNK * NW)
  pt = N_PAD // NS
  kern = pl.kernel(
      functools.partial(_sc_agg_body, N_PAD, k),
      out_type=jax.ShapeDtypeStruct((NC, N_PAD, D), jnp.float32),
      mesh=_sc_mesh(),
      scratch_types=[
          pltpu.VMEM((k * CHUNK,), jnp.int32),
          pltpu.VMEM((NBUF, CHUNK), jnp.int32),
          pltpu.VMEM((NBUF, CHUNK, D), jnp.float32),
          pltpu.VMEM((16, D), jnp.float32),
          pltpu.VMEM_SHARED((N_PAD, D), jnp.float32),
      ] + [pltpu.SemaphoreType.DMA] * (3 * NBUF),
  )
  return kern(g, src1, dst1)


ROWS = 2000


def _tc1_body(x_ref, w1_ref, degp_ref, g1_ref, dinv_ref):
  deg = degp_ref[0, :, :1] + degp_ref[1, :, :1] + 1.0
  dinv = lax.rsqrt(deg)
  h = jnp.dot(x_ref[...], w1_ref[...], preferred_element_type=jnp.float32)
  g1_ref[...] = h * dinv
  dinv_ref[...] = dinv


def _tc1(x, W1, degp):
  n, din = x.shape
  dh = W1.shape[1]
  grid = n // ROWS
  return pl.pallas_call(
      _tc1_body,
      grid=(grid,),
      in_specs=[
          pl.BlockSpec((ROWS, din), lambda i: (i, 0)),
          pl.BlockSpec((din, dh), lambda i: (0, 0)),
          pl.BlockSpec((NC, ROWS, DEGW), lambda i: (0, i, 0)),
      ],
      out_specs=[
          pl.BlockSpec((ROWS, dh), lambda i: (i, 0)),
          pl.BlockSpec((ROWS, 1), lambda i: (i, 0)),
      ],
      out_shape=[
          jax.ShapeDtypeStruct((n, dh), jnp.float32),
          jax.ShapeDtypeStruct((n, 1), jnp.float32),
      ],
  )(x, W1, degp)


def _tc2_body(a_ref, g1_ref, dinv_ref, b1_ref, w2_ref, g2_ref):
  dinv = dinv_ref[...]
  t = (a_ref[0] + a_ref[1] + g1_ref[...]) * dinv + b1_ref[...]
  r = jnp.maximum(t, 0.0)
  g2_ref[...] = jnp.dot(r, w2_ref[...],
                        preferred_element_type=jnp.float32) * dinv


def _tc2(a1, g1, dinv, b1, W2):
  n, dh = g1.shape
  dout = W2.shape[1]
  grid = n // ROWS
  return pl.pallas_call(
      _tc2_body,
      grid=(grid,),
      in_specs=[
          pl.BlockSpec((NC, ROWS, dh), lambda i: (0, i, 0)),
          pl.BlockSpec((ROWS, dh), lambda i: (i, 0)),
          pl.BlockSpec((ROWS, 1), lambda i: (i, 0)),
          pl.BlockSpec((1, dh), lambda i: (0, 0)),
          pl.BlockSpec((dh, dout), lambda i: (0, 0)),
      ],
      out_specs=pl.BlockSpec((ROWS, dout), lambda i: (i, 0)),
      out_shape=jax.ShapeDtypeStruct((n, dout), jnp.float32),
  )(a1, g1, dinv, b1, W2)


def _tc3_body(n_groups, a_ref, g2_ref, dinv_ref, b2_ref, batch_ref, h_ref,
              hs_ref):
  h = (a_ref[0] + a_ref[1] + g2_ref[...]) * dinv_ref[...] + b2_ref[...]
  h_ref[...] = h
  r = h.shape[0]
  onehot = (batch_ref[...] == lax.broadcasted_iota(
      jnp.int32, (r, n_groups), 1)).astype(jnp.float32)
  contrib = lax.dot_general(onehot, h, (((0,), (0,)), ((), ())),
                            preferred_element_type=jnp.float32)

  @pl.when(pl.program_id(0) == 0)
  def _():
    hs_ref[...] = jnp.zeros_like(hs_ref)

  hs_ref[...] += contrib


def _tc3(a2, g2, dinv, b2, batch2d, n_groups):
  n, dout = g2.shape
  grid = n // ROWS
  return pl.pallas_call(
      functools.partial(_tc3_body, n_groups),
      grid=(grid,),
      in_specs=[
          pl.BlockSpec((NC, ROWS, dout), lambda i: (0, i, 0)),
          pl.BlockSpec((ROWS, dout), lambda i: (i, 0)),
          pl.BlockSpec((ROWS, 1), lambda i: (i, 0)),
          pl.BlockSpec((1, dout), lambda i: (0, 0)),
          pl.BlockSpec((ROWS, 1), lambda i: (i, 0)),
      ],
      out_specs=[
          pl.BlockSpec((ROWS, dout), lambda i: (i, 0)),
          pl.BlockSpec((n_groups, dout), lambda i: (0, 0)),
      ],
      out_shape=[
          jax.ShapeDtypeStruct((n, dout), jnp.float32),
          jax.ShapeDtypeStruct((n_groups, dout), jnp.float32),
      ],
  )(a2, g2, dinv, b2, batch2d)


def _kernel_impl(x, edge_index, batch, W1, b1, W2, b2):
  n_groups = 64
  e = edge_index.shape[1]
  assert e % (CHUNK * NW) == 0
  k = e // (CHUNK * NW)
  src1 = edge_index[0]
  dst1 = edge_index[1]

  degp = _sc_deg(dst1)
  g1, dinv = _tc1(x, W1, degp)
  a1 = _sc_agg(g1, src1, dst1)
  g2 = _tc2(a1, g1, dinv, b1.reshape(1, -1), W2)
  a2 = _sc_agg(g2, src1, dst1)
  h, hs = _tc3(a2, g2, dinv, b2.reshape(1, -1), batch.reshape(-1, 1),
               n_groups)
  return (hs, h)


kernel = jax.jit(_kernel_impl)


# speedup vs baseline: 1.0446x; 1.0446x over previous
import functools

import jax
import jax.numpy as jnp
from jax import lax
from jax.experimental import pallas as pl
from jax.experimental.pallas import tpu as pltpu
from jax.experimental.pallas import tpu_sc as plsc

NC = 2
NS = 16
NW = NC * NS
LANES = 16
CHUNK = 40
NBUF = 5
N_PAD = 10240
DEGW = 32
D = 128


def _fill_rows(ref, n_rows, row_words, value):
  assert row_words % LANES == 0
  per_row = row_words // LANES

  def body(r, carry):
    for c in range(per_row):
      ref[r, pl.ds(c * LANES, LANES)] = jnp.full((LANES,), value, jnp.float32)
    return carry

  lax.fori_loop(0, n_rows, body, 0)


def _zero_shared(zbuf, acc_sh, s, pt, width):
  zr = zbuf.shape[0]
  _fill_rows(zbuf, zr, width, 0.0)
  for j in range(pt // zr):
    pltpu.sync_copy(zbuf, acc_sh.at[pl.ds(s * pt + j * zr, zr)])


def _sc_deg_body(n_pad, k, dst3_hbm, out_hbm, didx_all, ones_v, zbuf,
                 deg_sh, *ssems):
  c = lax.axis_index("c")
  s = lax.axis_index("s")
  w = s * NC + c
  pt = n_pad // NS

  _fill_rows(ones_v, CHUNK, DEGW, 1.0)
  _zero_shared(zbuf, deg_sh, s, pt, DEGW)
  pltpu.sync_copy(dst3_hbm.at[w], didx_all)
  plsc.subcore_barrier()

  assert k % NBUF == 0

  for b in range(NBUF):
    pltpu.async_copy(ones_v, deg_sh.at[didx_all.at[b]], ssems[b], add=True)

  def body(grp, carry):
    base = grp * NBUF
    for b in range(NBUF):
      pltpu.make_async_copy(
          ones_v, deg_sh.at[didx_all.at[base + b]], ssems[b]).wait()
      pltpu.async_copy(
          ones_v, deg_sh.at[didx_all.at[base + b]], ssems[b], add=True)
    return carry

  lax.fori_loop(1, k // NBUF, body, 0)
  for b in range(NBUF):
    pltpu.make_async_copy(
        ones_v, deg_sh.at[didx_all.at[b]], ssems[b]).wait()
  plsc.subcore_barrier()
  pltpu.sync_copy(deg_sh.at[pl.ds(s * pt, pt)],
                  out_hbm.at[c, pl.ds(s * pt, pt)])


def _sc_agg_body(n_pad, k, g_hbm, src1_hbm, dst1_hbm, out_hbm,
                 sidx_1d, didx, rows, zbuf, acc_sh, *sems):
  c = lax.axis_index("c")
  s = lax.axis_index("s")
  w = s * NC + c
  pt = n_pad // NS
  pe = k * CHUNK
  gsems = sems[:NBUF]
  ssems = sems[NBUF:2 * NBUF]
  isems = sems[2 * NBUF:]

  _zero_shared(zbuf, acc_sh, s, pt, D)
  pltpu.sync_copy(src1_hbm.at[pl.ds(w * pe, pe)], sidx_1d)
  plsc.subcore_barrier()

  assert k % NBUF == 0

  def fire(j, b):
    pltpu.async_copy(
        dst1_hbm.at[pl.ds(w * pe + j * CHUNK, CHUNK)], didx.at[b], isems[b])
    return pltpu.async_copy(
        g_hbm.at[sidx_1d.at[pl.ds(j * CHUNK, CHUNK)]], rows.at[b], gsems[b])

  def scatter(j, b, gd):
    gd.wait()
    pltpu.make_async_copy(
        dst1_hbm.at[pl.ds(w * pe + j * CHUNK, CHUNK)],
        didx.at[b], isems[b]).wait()
    pltpu.async_copy(
        rows.at[b], acc_sh.at[didx.at[b]], ssems[b], add=True)

  gds = [fire(b, b) for b in range(NBUF)]
  for b in range(NBUF):
    scatter(b, b, gds[b])

  def body(grp, carry):
    base = grp * NBUF
    gds = []
    for b in range(NBUF):
      pltpu.make_async_copy(
          rows.at[b], acc_sh.at[didx.at[b]], ssems[b]).wait()
      gds.append(fire(base + b, b))
    for b in range(NBUF):
      scatter(base + b, b, gds[b])
    return carry

  lax.fori_loop(1, k // NBUF, body, 0)
  for b in range(NBUF):
    pltpu.make_async_copy(
        rows.at[b], acc_sh.at[didx.at[b]], ssems[b]).wait()
  plsc.subcore_barrier()
  pltpu.sync_copy(acc_sh.at[pl.ds(s * pt, pt)],
                  out_hbm.at[c, pl.ds(s * pt, pt)])


def _sc_mesh():
  return plsc.VectorSubcoreMesh(core_axis_name="c", subcore_axis_name="s",
                                num_cores=NC, num_subcores=NS)


def _sc_deg(dst3):
  k = dst3.shape[1]
  pt = N_PAD // NS
  kern = pl.kernel(
      functools.partial(_sc_deg_body, N_PAD, k),
      out_type=jax.ShapeDtypeStruct((NC, N_PAD, DEGW), jnp.float32),
      mesh=_sc_mesh(),
      scratch_types=[
          pltpu.VMEM((k, CHUNK), jnp.int32),
          pltpu.VMEM((CHUNK, DEGW), jnp.float32),
          pltpu.VMEM((pt // 10, DEGW), jnp.float32),
          pltpu.VMEM_SHARED((N_PAD, DEGW), jnp.float32),
      ] + [pltpu.SemaphoreType.DMA] * NBUF,
      compiler_params=pltpu.CompilerParams(use_tc_tiling_on_sc=False),
  )
  return kern(dst3)


def _sc_agg(g, src1, dst1):
  n = g.shape[0]
  e = src1.shape[0]
  assert n <= N_PAD and g.shape[1] == D and e % (CHUNK * NW) == 0
  k = e // (CHUNK * NW)
  pt = N_PAD // NS
  kern = pl.kernel(
      functools.partial(_sc_agg_body, N_PAD, k),
      out_type=jax.ShapeDtypeStruct((NC, N_PAD, D), jnp.float32),
      mesh=_sc_mesh(),
      scratch_types=[
          pltpu.VMEM((k * CHUNK,), jnp.int32),
          pltpu.VMEM((NBUF, CHUNK), jnp.int32),
          pltpu.VMEM((NBUF, CHUNK, D), jnp.float32),
          pltpu.VMEM((16, D), jnp.float32),
          pltpu.VMEM_SHARED((N_PAD, D), jnp.float32),
      ] + [pltpu.SemaphoreType.DMA] * (3 * NBUF),
  )
  return kern(g, src1, dst1)


ROWS = 2000


def _tc1_body(x_ref, w1_ref, degp_ref, src_ref, g1_ref, dinv_ref, src1_ref):
  deg = degp_ref[0, :, :1] + degp_ref[1, :, :1] + 1.0
  dinv = lax.rsqrt(deg)
  h = jnp.dot(x_ref[...], w1_ref[...], preferred_element_type=jnp.float32)
  g1_ref[...] = h * dinv
  dinv_ref[...] = dinv

  @pl.when(pl.program_id(0) == 0)
  def _():
    src1_ref[...] = src_ref[0]


def _tc1(x, W1, degp, edge_index):
  n, din = x.shape
  dh = W1.shape[1]
  e = edge_index.shape[1]
  grid = n // ROWS
  eb = e // grid
  return pl.pallas_call(
      _tc1_body,
      grid=(grid,),
      in_specs=[
          pl.BlockSpec((ROWS, din), lambda i: (i, 0)),
          pl.BlockSpec((din, dh), lambda i: (0, 0)),
          pl.BlockSpec((NC, ROWS, DEGW), lambda i: (0, i, 0)),
          pl.BlockSpec((2, e), lambda i: (0, 0)),
      ],
      out_specs=[
          pl.BlockSpec((ROWS, dh), lambda i: (i, 0)),
          pl.BlockSpec((ROWS, 1), lambda i: (i, 0)),
          pl.BlockSpec((e,), lambda i: (0,)),
      ],
      out_shape=[
          jax.ShapeDtypeStruct((n, dh), jnp.float32),
          jax.ShapeDtypeStruct((n, 1), jnp.float32),
          jax.ShapeDtypeStruct((e,), jnp.int32),
      ],
  )(x, W1, degp, edge_index)


def _tc2_body(a_ref, g1_ref, dinv_ref, b1_ref, w2_ref, g2_ref):
  dinv = dinv_ref[...]
  t = (a_ref[0] + a_ref[1] + g1_ref[...]) * dinv + b1_ref[...]
  r = jnp.maximum(t, 0.0)
  g2_ref[...] = jnp.dot(r, w2_ref[...],
                        preferred_element_type=jnp.float32) * dinv


def _tc2(a1, g1, dinv, b1, W2):
  n, dh = g1.shape
  dout = W2.shape[1]
  grid = n // ROWS
  return pl.pallas_call(
      _tc2_body,
      grid=(grid,),
      in_specs=[
          pl.BlockSpec((NC, ROWS, dh), lambda i: (0, i, 0)),
          pl.BlockSpec((ROWS, dh), lambda i: (i, 0)),
          pl.BlockSpec((ROWS, 1), lambda i: (i, 0)),
          pl.BlockSpec((1, dh), lambda i: (0, 0)),
          pl.BlockSpec((dh, dout), lambda i: (0, 0)),
      ],
      out_specs=pl.BlockSpec((ROWS, dout), lambda i: (i, 0)),
      out_shape=jax.ShapeDtypeStruct((n, dout), jnp.float32),
  )(a1, g1, dinv, b1, W2)


def _tc3_body(n_groups, a_ref, g2_ref, dinv_ref, b2_ref, batch_ref, h_ref,
              hs_ref):
  h = (a_ref[0] + a_ref[1] + g2_ref[...]) * dinv_ref[...] + b2_ref[...]
  h_ref[...] = h
  r = h.shape[0]
  onehot = (batch_ref[...] == lax.broadcasted_iota(
      jnp.int32, (r, n_groups), 1)).astype(jnp.float32)
  contrib = lax.dot_general(onehot, h, (((0,), (0,)), ((), ())),
                            preferred_element_type=jnp.float32)

  @pl.when(pl.program_id(0) == 0)
  def _():
    hs_ref[...] = jnp.zeros_like(hs_ref)

  hs_ref[...] += contrib


def _tc3(a2, g2, dinv, b2, batch2d, n_groups):
  n, dout = g2.shape
  grid = n // ROWS
  return pl.pallas_call(
      functools.partial(_tc3_body, n_groups),
      grid=(grid,),
      in_specs=[
          pl.BlockSpec((NC, ROWS, dout), lambda i: (0, i, 0)),
          pl.BlockSpec((ROWS, dout), lambda i: (i, 0)),
          pl.BlockSpec((ROWS, 1), lambda i: (i, 0)),
          pl.BlockSpec((1, dout), lambda i: (0, 0)),
          pl.BlockSpec((ROWS, 1), lambda i: (i, 0)),
      ],
      out_specs=[
          pl.BlockSpec((ROWS, dout), lambda i: (i, 0)),
          pl.BlockSpec((n_groups, dout), lambda i: (0, 0)),
      ],
      out_shape=[
          jax.ShapeDtypeStruct((n, dout), jnp.float32),
          jax.ShapeDtypeStruct((n_groups, dout), jnp.float32),
      ],
  )(a2, g2, dinv, b2, batch2d)


def _kernel_impl(x, edge_index, batch, W1, b1, W2, b2):
  n_groups = 64
  e = edge_index.shape[1]
  assert e % (CHUNK * NW) == 0
  k = e // (CHUNK * NW)
  dst1 = edge_index[1]
  dst3 = dst1.reshape(NW, k, CHUNK)

  degp = _sc_deg(dst3)
  g1, dinv, src1 = _tc1(x, W1, degp, edge_index)
  a1 = _sc_agg(g1, src1, dst1)
  g2 = _tc2(a1, g1, dinv, b1.reshape(1, -1), W2)
  a2 = _sc_agg(g2, src1, dst1)
  h, hs = _tc3(a2, g2, dinv, b2.reshape(1, -1), batch.reshape(-1, 1),
               n_groups)
  return (hs, h)


kernel = jax.jit(_kernel_impl)


# speedup vs baseline: 1.0926x; 1.0460x over previous
import functools

import jax
import jax.numpy as jnp
from jax import lax
from jax.experimental import pallas as pl
from jax.experimental.pallas import tpu as pltpu
from jax.experimental.pallas import tpu_sc as plsc

NC = 2
NS = 16
NW = NC * NS
LANES = 16
CHUNK = 40
NBUF = 5
N_PAD = 10240
DEGW = 32
D = 128


def _fill_rows(ref, n_rows, row_words, value):
  assert row_words % LANES == 0
  per_row = row_words // LANES

  def body(r, carry):
    for c in range(per_row):
      ref[r, pl.ds(c * LANES, LANES)] = jnp.full((LANES,), value, jnp.float32)
    return carry

  lax.fori_loop(0, n_rows, body, 0)


def _zero_shared(zbuf, acc_sh, s, pt, width):
  zr = zbuf.shape[0]
  _fill_rows(zbuf, zr, width, 0.0)
  for j in range(pt // zr):
    pltpu.sync_copy(zbuf, acc_sh.at[pl.ds(s * pt + j * zr, zr)])


def _sc_deg_body(n_pad, k, dst3_hbm, out_hbm, didx_all, ones_v, zbuf,
                 deg_sh, *ssems):
  c = lax.axis_index("c")
  s = lax.axis_index("s")
  w = s * NC + c
  pt = n_pad // NS

  _fill_rows(ones_v, CHUNK, DEGW, 1.0)
  _zero_shared(zbuf, deg_sh, s, pt, DEGW)
  pltpu.sync_copy(dst3_hbm.at[w], didx_all)
  plsc.subcore_barrier()

  assert k % NBUF == 0

  for b in range(NBUF):
    pltpu.async_copy(ones_v, deg_sh.at[didx_all.at[b]], ssems[b], add=True)

  def body(grp, carry):
    base = grp * NBUF
    for b in range(NBUF):
      pltpu.make_async_copy(
          ones_v, deg_sh.at[didx_all.at[base + b]], ssems[b]).wait()
      pltpu.async_copy(
          ones_v, deg_sh.at[didx_all.at[base + b]], ssems[b], add=True)
    return carry

  lax.fori_loop(1, k // NBUF, body, 0)
  for b in range(NBUF):
    pltpu.make_async_copy(
        ones_v, deg_sh.at[didx_all.at[b]], ssems[b]).wait()
  plsc.subcore_barrier()
  pltpu.sync_copy(deg_sh.at[pl.ds(s * pt, pt)],
                  out_hbm.at[c, pl.ds(s * pt, pt)])


def _sc_agg_body(n_pad, k, g_hbm, src1_hbm, dst1_hbm, out_hbm,
                 sidx_1d, didx, rows, zbuf, acc_sh, *sems):
  c = lax.axis_index("c")
  s = lax.axis_index("s")
  w = s * NC + c
  pt = n_pad // NS
  pe = k * CHUNK
  gsems = sems[:NBUF]
  ssems = sems[NBUF:2 * NBUF]
  isems = sems[2 * NBUF:]

  _zero_shared(zbuf, acc_sh, s, pt, D)
  pltpu.sync_copy(src1_hbm.at[pl.ds(w * pe, pe)], sidx_1d)
  plsc.subcore_barrier()

  assert k % NBUF == 0

  def fire(j, b):
    pltpu.async_copy(
        dst1_hbm.at[pl.ds(w * pe + j * CHUNK, CHUNK)], didx.at[b], isems[b])
    return pltpu.async_copy(
        g_hbm.at[sidx_1d.at[pl.ds(j * CHUNK, CHUNK)]], rows.at[b], gsems[b])

  def scatter(j, b, gd):
    gd.wait()
    pltpu.make_async_copy(
        dst1_hbm.at[pl.ds(w * pe + j * CHUNK, CHUNK)],
        didx.at[b], isems[b]).wait()
    pltpu.async_copy(
        rows.at[b], acc_sh.at[didx.at[b]], ssems[b], add=True)

  gds = [fire(b, b) for b in range(NBUF)]
  for b in range(NBUF):
    scatter(b, b, gds[b])

  def body(grp, carry):
    base = grp * NBUF
    gds = []
    for b in range(NBUF):
      pltpu.make_async_copy(
          rows.at[b], acc_sh.at[didx.at[b]], ssems[b]).wait()
      gds.append(fire(base + b, b))
    for b in range(NBUF):
      scatter(base + b, b, gds[b])
    return carry

  lax.fori_loop(1, k // NBUF, body, 0)
  for b in range(NBUF):
    pltpu.make_async_copy(
        rows.at[b], acc_sh.at[didx.at[b]], ssems[b]).wait()
  plsc.subcore_barrier()
  pltpu.sync_copy(acc_sh.at[pl.ds(s * pt, pt)],
                  out_hbm.at[c, pl.ds(s * pt, pt)])


def _sc_mesh():
  return plsc.VectorSubcoreMesh(core_axis_name="c", subcore_axis_name="s",
                                num_cores=NC, num_subcores=NS)


def _sc_deg(dst3):
  k = dst3.shape[1]
  pt = N_PAD // NS
  kern = pl.kernel(
      functools.partial(_sc_deg_body, N_PAD, k),
      out_type=jax.ShapeDtypeStruct((NC, N_PAD, DEGW), jnp.float32),
      mesh=_sc_mesh(),
      scratch_types=[
          pltpu.VMEM((k, CHUNK), jnp.int32),
          pltpu.VMEM((CHUNK, DEGW), jnp.float32),
          pltpu.VMEM((pt // 10, DEGW), jnp.float32),
          pltpu.VMEM_SHARED((N_PAD, DEGW), jnp.float32),
      ] + [pltpu.SemaphoreType.DMA] * NBUF,
      compiler_params=pltpu.CompilerParams(use_tc_tiling_on_sc=False),
  )
  return kern(dst3)


def _sc_agg(g, src1, dst1):
  n = g.shape[0]
  e = src1.shape[0]
  assert n <= N_PAD and g.shape[1] == D and e % (CHUNK * NW) == 0
  k = e // (CHUNK * NW)
  pt = N_PAD // NS
  kern = pl.kernel(
      functools.partial(_sc_agg_body, N_PAD, k),
      out_type=jax.ShapeDtypeStruct((NC, N_PAD, D), jnp.float32),
      mesh=_sc_mesh(),
      scratch_types=[
          pltpu.VMEM((k * CHUNK,), jnp.int32),
          pltpu.VMEM((NBUF, CHUNK), jnp.int32),
          pltpu.VMEM((NBUF, CHUNK, D), jnp.float32),
          pltpu.VMEM((16, D), jnp.float32),
          pltpu.VMEM_SHARED((N_PAD, D), jnp.float32),
      ] + [pltpu.SemaphoreType.DMA] * (3 * NBUF),
  )
  return kern(g, src1, dst1)


ROWS = 2000


def _tc0_body(e_ref, src1_ref, dst1_ref):
  src1_ref[...] = e_ref[0]
  dst1_ref[...] = e_ref[1]


def _tc0(edge_index):
  e = edge_index.shape[1]
  return pl.pallas_call(
      _tc0_body,
      out_shape=[
          jax.ShapeDtypeStruct((e,), jnp.int32),
          jax.ShapeDtypeStruct((e,), jnp.int32),
      ],
  )(edge_index)


def _tc1_body(x_ref, w1_ref, degp_ref, g1_ref, dinv_ref):
  deg = degp_ref[0, :, :1] + degp_ref[1, :, :1] + 1.0
  dinv = lax.rsqrt(deg)
  h = jnp.dot(x_ref[...], w1_ref[...], preferred_element_type=jnp.float32)
  g1_ref[...] = h * dinv
  dinv_ref[...] = dinv


def _tc1(x, W1, degp):
  n, din = x.shape
  dh = W1.shape[1]
  grid = n // ROWS
  return pl.pallas_call(
      _tc1_body,
      grid=(grid,),
      in_specs=[
          pl.BlockSpec((ROWS, din), lambda i: (i, 0)),
          pl.BlockSpec((din, dh), lambda i: (0, 0)),
          pl.BlockSpec((NC, ROWS, DEGW), lambda i: (0, i, 0)),
      ],
      out_specs=[
          pl.BlockSpec((ROWS, dh), lambda i: (i, 0)),
          pl.BlockSpec((ROWS, 1), lambda i: (i, 0)),
      ],
      out_shape=[
          jax.ShapeDtypeStruct((n, dh), jnp.float32),
          jax.ShapeDtypeStruct((n, 1), jnp.float32),
      ],
  )(x, W1, degp)


def _tc2_body(a_ref, g1_ref, dinv_ref, b1_ref, w2_ref, g2_ref):
  dinv = dinv_ref[...]
  t = (a_ref[0] + a_ref[1] + g1_ref[...]) * dinv + b1_ref[...]
  r = jnp.maximum(t, 0.0)
  g2_ref[...] = jnp.dot(r, w2_ref[...],
                        preferred_element_type=jnp.float32) * dinv


def _tc2(a1, g1, dinv, b1, W2):
  n, dh = g1.shape
  dout = W2.shape[1]
  grid = n // ROWS
  return pl.pallas_call(
      _tc2_body,
      grid=(grid,),
      in_specs=[
          pl.BlockSpec((NC, ROWS, dh), lambda i: (0, i, 0)),
          pl.BlockSpec((ROWS, dh), lambda i: (i, 0)),
          pl.BlockSpec((ROWS, 1), lambda i: (i, 0)),
          pl.BlockSpec((1, dh), lambda i: (0, 0)),
          pl.BlockSpec((dh, dout), lambda i: (0, 0)),
      ],
      out_specs=pl.BlockSpec((ROWS, dout), lambda i: (i, 0)),
      out_shape=jax.ShapeDtypeStruct((n, dout), jnp.float32),
  )(a1, g1, dinv, b1, W2)


def _tc3_body(n_groups, a_ref, g2_ref, dinv_ref, b2_ref, batch_ref, h_ref,
              hs_ref):
  h = (a_ref[0] + a_ref[1] + g2_ref[...]) * dinv_ref[...] + b2_ref[...]
  h_ref[...] = h
  r = h.shape[0]
  onehot = (batch_ref[...] == lax.broadcasted_iota(
      jnp.int32, (r, n_groups), 1)).astype(jnp.float32)
  contrib = lax.dot_general(onehot, h, (((0,), (0,)), ((), ())),
                            preferred_element_type=jnp.float32)

  @pl.when(pl.program_id(0) == 0)
  def _():
    hs_ref[...] = jnp.zeros_like(hs_ref)

  hs_ref[...] += contrib


def _tc3(a2, g2, dinv, b2, batch2d, n_groups):
  n, dout = g2.shape
  grid = n // ROWS
  return pl.pallas_call(
      functools.partial(_tc3_body, n_groups),
      grid=(grid,),
      in_specs=[
          pl.BlockSpec((NC, ROWS, dout), lambda i: (0, i, 0)),
          pl.BlockSpec((ROWS, dout), lambda i: (i, 0)),
          pl.BlockSpec((ROWS, 1), lambda i: (i, 0)),
          pl.BlockSpec((1, dout), lambda i: (0, 0)),
          pl.BlockSpec((ROWS, 1), lambda i: (i, 0)),
      ],
      out_specs=[
          pl.BlockSpec((ROWS, dout), lambda i: (i, 0)),
          pl.BlockSpec((n_groups, dout), lambda i: (0, 0)),
      ],
      out_shape=[
          jax.ShapeDtypeStruct((n, dout), jnp.float32),
          jax.ShapeDtypeStruct((n_groups, dout), jnp.float32),
      ],
  )(a2, g2, dinv, b2, batch2d)


def _kernel_impl(x, edge_index, batch, W1, b1, W2, b2):
  n_groups = 64
  e = edge_index.shape[1]
  assert e % (CHUNK * NW) == 0
  k = e // (CHUNK * NW)
  src1, dst1 = _tc0(edge_index)
  dst3 = dst1.reshape(NW, k, CHUNK)

  degp = _sc_deg(dst3)
  g1, dinv = _tc1(x, W1, degp)
  a1 = _sc_agg(g1, src1, dst1)
  g2 = _tc2(a1, g1, dinv, b1.reshape(1, -1), W2)
  a2 = _sc_agg(g2, src1, dst1)
  h, hs = _tc3(a2, g2, dinv, b2.reshape(1, -1), batch.reshape(-1, 1),
               n_groups)
  return (hs, h)


kernel = jax.jit(_kernel_impl)


# speedup vs baseline: 1.1240x; 1.0287x over previous
import functools

import jax
import jax.numpy as jnp
from jax import lax
from jax.experimental import pallas as pl
from jax.experimental.pallas import tpu as pltpu
from jax.experimental.pallas import tpu_sc as plsc

NC = 2
NS = 16
NW = NC * NS
LANES = 16
CHUNK = 40
NBUF = 5
N_PAD = 10240
DEGW = 16
D = 128


def _fill_rows(ref, n_rows, row_words, value):
  assert row_words % LANES == 0
  per_row = row_words // LANES

  def body(r, carry):
    for c in range(per_row):
      ref[r, pl.ds(c * LANES, LANES)] = jnp.full((LANES,), value, jnp.float32)
    return carry

  lax.fori_loop(0, n_rows, body, 0)


def _zero_shared(zbuf, acc_sh, s, pt, width):
  zr = zbuf.shape[0]
  _fill_rows(zbuf, zr, width, 0.0)
  for j in range(pt // zr):
    pltpu.sync_copy(zbuf, acc_sh.at[pl.ds(s * pt + j * zr, zr)])


def _sc_deg_body(n_pad, k, dst3_hbm, out_hbm, didx_all, ones_v, zbuf,
                 deg_sh, *ssems):
  c = lax.axis_index("c")
  s = lax.axis_index("s")
  w = s * NC + c
  pt = n_pad // NS

  _fill_rows(ones_v, CHUNK, DEGW, 1.0)
  _zero_shared(zbuf, deg_sh, s, pt, DEGW)
  pltpu.sync_copy(dst3_hbm.at[w], didx_all)
  plsc.subcore_barrier()

  assert k % NBUF == 0

  for b in range(NBUF):
    pltpu.async_copy(ones_v, deg_sh.at[didx_all.at[b]], ssems[b], add=True)

  def body(grp, carry):
    base = grp * NBUF
    for b in range(NBUF):
      pltpu.make_async_copy(
          ones_v, deg_sh.at[didx_all.at[base + b]], ssems[b]).wait()
      pltpu.async_copy(
          ones_v, deg_sh.at[didx_all.at[base + b]], ssems[b], add=True)
    return carry

  lax.fori_loop(1, k // NBUF, body, 0)
  for b in range(NBUF):
    pltpu.make_async_copy(
        ones_v, deg_sh.at[didx_all.at[b]], ssems[b]).wait()
  plsc.subcore_barrier()
  pltpu.sync_copy(deg_sh.at[pl.ds(s * pt, pt)],
                  out_hbm.at[c, pl.ds(s * pt, pt)])


def _sc_agg_body(n_pad, k, g_hbm, src1_hbm, dst1_hbm, out_hbm,
                 sidx_1d, didx, rows, zbuf, acc_sh, *sems):
  c = lax.axis_index("c")
  s = lax.axis_index("s")
  w = s * NC + c
  pt = n_pad // NS
  pe = k * CHUNK
  gsems = sems[:NBUF]
  ssems = sems[NBUF:2 * NBUF]
  isems = sems[2 * NBUF:]

  _zero_shared(zbuf, acc_sh, s, pt, D)
  pltpu.sync_copy(src1_hbm.at[pl.ds(w * pe, pe)], sidx_1d)
  plsc.subcore_barrier()

  assert k % NBUF == 0

  def fire(j, b):
    pltpu.async_copy(
        dst1_hbm.at[pl.ds(w * pe + j * CHUNK, CHUNK)], didx.at[b], isems[b])
    return pltpu.async_copy(
        g_hbm.at[sidx_1d.at[pl.ds(j * CHUNK, CHUNK)]], rows.at[b], gsems[b])

  def scatter(j, b, gd):
    gd.wait()
    pltpu.make_async_copy(
        dst1_hbm.at[pl.ds(w * pe + j * CHUNK, CHUNK)],
        didx.at[b], isems[b]).wait()
    pltpu.async_copy(
        rows.at[b], acc_sh.at[didx.at[b]], ssems[b], add=True)

  gds = [fire(b, b) for b in range(NBUF)]
  for b in range(NBUF):
    scatter(b, b, gds[b])

  def body(grp, carry):
    base = grp * NBUF
    gds = []
    for b in range(NBUF):
      pltpu.make_async_copy(
          rows.at[b], acc_sh.at[didx.at[b]], ssems[b]).wait()
      gds.append(fire(base + b, b))
    for b in range(NBUF):
      scatter(base + b, b, gds[b])
    return carry

  lax.fori_loop(1, k // NBUF, body, 0)
  for b in range(NBUF):
    pltpu.make_async_copy(
        rows.at[b], acc_sh.at[didx.at[b]], ssems[b]).wait()
  plsc.subcore_barrier()
  pltpu.sync_copy(acc_sh.at[pl.ds(s * pt, pt)],
                  out_hbm.at[c, pl.ds(s * pt, pt)])


def _sc_mesh():
  return plsc.VectorSubcoreMesh(core_axis_name="c", subcore_axis_name="s",
                                num_cores=NC, num_subcores=NS)


def _sc_deg(dst3):
  k = dst3.shape[1]
  pt = N_PAD // NS
  kern = pl.kernel(
      functools.partial(_sc_deg_body, N_PAD, k),
      out_type=jax.ShapeDtypeStruct((NC, N_PAD, DEGW), jnp.float32),
      mesh=_sc_mesh(),
      scratch_types=[
          pltpu.VMEM((k, CHUNK), jnp.int32),
          pltpu.VMEM((CHUNK, DEGW), jnp.float32),
          pltpu.VMEM((pt // 10, DEGW), jnp.float32),
          pltpu.VMEM_SHARED((N_PAD, DEGW), jnp.float32),
      ] + [pltpu.SemaphoreType.DMA] * NBUF,
      compiler_params=pltpu.CompilerParams(use_tc_tiling_on_sc=False),
  )
  return kern(dst3)


def _sc_agg(g, src1, dst1):
  n = g.shape[0]
  e = src1.shape[0]
  assert n <= N_PAD and g.shape[1] == D and e % (CHUNK * NW) == 0
  k = e // (CHUNK * NW)
  pt = N_PAD // NS
  kern = pl.kernel(
      functools.partial(_sc_agg_body, N_PAD, k),
      out_type=jax.ShapeDtypeStruct((NC, N_PAD, D), jnp.float32),
      mesh=_sc_mesh(),
      scratch_types=[
          pltpu.VMEM((k * CHUNK,), jnp.int32),
          pltpu.VMEM((NBUF, CHUNK), jnp.int32),
          pltpu.VMEM((NBUF, CHUNK, D), jnp.float32),
          pltpu.VMEM((16, D), jnp.float32),
          pltpu.VMEM_SHARED((N_PAD, D), jnp.float32),
      ] + [pltpu.SemaphoreType.DMA] * (3 * NBUF),
  )
  return kern(g, src1, dst1)


ROWS = 2000


def _tc0_body(e_ref, src1_ref, dst1_ref):
  src1_ref[...] = e_ref[0]
  dst1_ref[...] = e_ref[1]


def _tc0(edge_index):
  e = edge_index.shape[1]
  return pl.pallas_call(
      _tc0_body,
      out_shape=[
          jax.ShapeDtypeStruct((e,), jnp.int32),
          jax.ShapeDtypeStruct((e,), jnp.int32),
      ],
  )(edge_index)


def _tc1_body(x_ref, w1_ref, degp_ref, g1_ref, dinv_ref):
  deg = degp_ref[0, :, :1] + degp_ref[1, :, :1] + 1.0
  dinv = lax.rsqrt(deg)
  h = jnp.dot(x_ref[...], w1_ref[...], preferred_element_type=jnp.float32)
  g1_ref[...] = h * dinv
  dinv_ref[...] = dinv


def _tc1(x, W1, degp):
  n, din = x.shape
  dh = W1.shape[1]
  grid = n // ROWS
  return pl.pallas_call(
      _tc1_body,
      grid=(grid,),
      in_specs=[
          pl.BlockSpec((ROWS, din), lambda i: (i, 0)),
          pl.BlockSpec((din, dh), lambda i: (0, 0)),
          pl.BlockSpec((NC, ROWS, DEGW), lambda i: (0, i, 0)),
      ],
      out_specs=[
          pl.BlockSpec((ROWS, dh), lambda i: (i, 0)),
          pl.BlockSpec((ROWS, 1), lambda i: (i, 0)),
      ],
      out_shape=[
          jax.ShapeDtypeStruct((n, dh), jnp.float32),
          jax.ShapeDtypeStruct((n, 1), jnp.float32),
      ],
  )(x, W1, degp)


def _tc2_body(a_ref, g1_ref, dinv_ref, b1_ref, w2_ref, g2_ref):
  dinv = dinv_ref[...]
  t = (a_ref[0] + a_ref[1] + g1_ref[...]) * dinv + b1_ref[...]
  r = jnp.maximum(t, 0.0)
  g2_ref[...] = jnp.dot(r, w2_ref[...],
                        preferred_element_type=jnp.float32) * dinv


def _tc2(a1, g1, dinv, b1, W2):
  n, dh = g1.shape
  dout = W2.shape[1]
  grid = n // ROWS
  return pl.pallas_call(
      _tc2_body,
      grid=(grid,),
      in_specs=[
          pl.BlockSpec((NC, ROWS, dh), lambda i: (0, i, 0)),
          pl.BlockSpec((ROWS, dh), lambda i: (i, 0)),
          pl.BlockSpec((ROWS, 1), lambda i: (i, 0)),
          pl.BlockSpec((1, dh), lambda i: (0, 0)),
          pl.BlockSpec((dh, dout), lambda i: (0, 0)),
      ],
      out_specs=pl.BlockSpec((ROWS, dout), lambda i: (i, 0)),
      out_shape=jax.ShapeDtypeStruct((n, dout), jnp.float32),
  )(a1, g1, dinv, b1, W2)


def _tc3_body(n_groups, a_ref, g2_ref, dinv_ref, b2_ref, batch_ref, h_ref,
              hs_ref):
  h = (a_ref[0] + a_ref[1] + g2_ref[...]) * dinv_ref[...] + b2_ref[...]
  h_ref[...] = h
  r = h.shape[0]
  onehot = (batch_ref[...] == lax.broadcasted_iota(
      jnp.int32, (r, n_groups), 1)).astype(jnp.float32)
  contrib = lax.dot_general(onehot, h, (((0,), (0,)), ((), ())),
                            preferred_element_type=jnp.float32)

  @pl.when(pl.program_id(0) == 0)
  def _():
    hs_ref[...] = jnp.zeros_like(hs_ref)

  hs_ref[...] += contrib


def _tc3(a2, g2, dinv, b2, batch2d, n_groups):
  n, dout = g2.shape
  grid = n // ROWS
  return pl.pallas_call(
      functools.partial(_tc3_body, n_groups),
      grid=(grid,),
      in_specs=[
          pl.BlockSpec((NC, ROWS, dout), lambda i: (0, i, 0)),
          pl.BlockSpec((ROWS, dout), lambda i: (i, 0)),
          pl.BlockSpec((ROWS, 1), lambda i: (i, 0)),
          pl.BlockSpec((1, dout), lambda i: (0, 0)),
          pl.BlockSpec((ROWS, 1), lambda i: (i, 0)),
      ],
      out_specs=[
          pl.BlockSpec((ROWS, dout), lambda i: (i, 0)),
          pl.BlockSpec((n_groups, dout), lambda i: (0, 0)),
      ],
      out_shape=[
          jax.ShapeDtypeStruct((n, dout), jnp.float32),
          jax.ShapeDtypeStruct((n_groups, dout), jnp.float32),
      ],
  )(a2, g2, dinv, b2, batch2d)


def _kernel_impl(x, edge_index, batch, W1, b1, W2, b2):
  n_groups = 64
  e = edge_index.shape[1]
  assert e % (CHUNK * NW) == 0
  k = e // (CHUNK * NW)
  src1, dst1 = _tc0(edge_index)
  dst3 = dst1.reshape(NW, k, CHUNK)

  degp = _sc_deg(dst3)
  g1, dinv = _tc1(x, W1, degp)
  a1 = _sc_agg(g1, src1, dst1)
  g2 = _tc2(a1, g1, dinv, b1.reshape(1, -1), W2)
  a2 = _sc_agg(g2, src1, dst1)
  h, hs = _tc3(a2, g2, dinv, b2.reshape(1, -1), batch.reshape(-1, 1),
               n_groups)
  return (hs, h)


kernel = jax.jit(_kernel_impl)
